# baseline (speedup 1.0000x reference)
import functools

import jax
import jax.numpy as jnp
from jax import lax
from jax.experimental import pallas as pl
from jax.experimental.pallas import tpu as pltpu
from jax.experimental.pallas import tpu_sc as plsc

_N = 1024
_B = 8
_KNN_K = 5
_ALPHA = 1.05
_W1 = 5.0
_W2 = 3.0
_BIG = 3.0e38
_NW = 32
_RPW = (_B * _N) // _NW
_QPB = _N // _RPW
_NC = 64


_GDN = jax.lax.GatherDimensionNumbers(
    offset_dims=(), collapsed_slice_dims=(0,), start_index_map=(0,))


def _shuffle(x, perm):
    return lax.gather(x, perm[:, None], _GDN, (1,),
                      mode=lax.GatherScatterMode.PROMISE_IN_BOUNDS)


def _hmin(x, iot):
    for sh in (1, 2, 4, 8):
        x = jnp.minimum(x, _shuffle(x, iot ^ sh))
    return x


def _sc_body(ax_h, ay_h, az_h, ox_h, oy_h, oz_h, val_h, cham_h,
             axv, ayv, azv, oxv, oyv, ozv, valv, chamv):
    wid = lax.axis_index("s") * 2 + lax.axis_index("c")
    b = wid // _QPB
    base = (wid % _QPB) * _RPW

    pltpu.sync_copy(ax_h.at[b], axv.at[pl.ds(0, _N)])
    pltpu.sync_copy(ay_h.at[b], ayv.at[pl.ds(0, _N)])
    pltpu.sync_copy(az_h.at[b], azv.at[pl.ds(0, _N)])
    pltpu.sync_copy(ox_h.at[b], oxv)
    pltpu.sync_copy(oy_h.at[b], oyv)
    pltpu.sync_copy(oz_h.at[b], ozv)

    iot = lax.iota(jnp.int32, 16)

    def row(t, carry):
        i = _RPW - 1 - t
        n = base + i
        bx = jnp.full((16,), axv[pl.ds(n, 16)][0], jnp.float32)
        by = jnp.full((16,), ayv[pl.ds(n, 16)][0], jnp.float32)
        bz = jnp.full((16,), azv[pl.ds(n, 16)][0], jnp.float32)
        cmin = jnp.full((16,), _BIG, jnp.float32)
        R = [jnp.full((16,), _BIG, jnp.float32) for _ in range(6)]
        for c in range(_NC):
            sl = pl.ds(c * 16, 16)
            dx = oxv[sl] - bx
            dy = oyv[sl] - by
            dz = ozv[sl] - bz
            cmin = jnp.minimum(cmin, dx * dx + dy * dy + dz * dz)
            ex = axv[sl] - bx
            ey = ayv[sl] - by
            ez = azv[sl] - bz
            x = ex * ex + ey * ey + ez * ez
            for j in range(5):
                mj = jnp.minimum(R[j], x)
                x = jnp.maximum(R[j], x)
                R[j] = mj
            R[5] = jnp.minimum(R[5], x)

        chamv[pl.ds(i + 113, 16)] = _hmin(cmin, iot)

        acc = jnp.zeros((16,), jnp.float32)
        bigv = jnp.full((16,), _BIG, jnp.float32)
        i16 = jnp.full((16,), 16, jnp.int32)
        for j in range(_KNN_K + 1):
            sv = _hmin(R[0], iot)
            if j > 0:
                acc = acc + sv
            if j < _KNN_K:
                eq = R[0] == sv
                slot = _hmin(jnp.where(eq, iot, i16), iot)
                onehot = iot == slot
                for t in range(5):
                    R[t] = jnp.where(onehot, R[t + 1], R[t])
                R[5] = jnp.where(onehot, bigv, R[5])
        valv[pl.ds(i + 113, 16)] = acc / jnp.float32(_KNN_K)
        return carry

    lax.fori_loop(0, _RPW, row, 0)

    pltpu.sync_copy(valv.at[pl.ds(128, _RPW)], val_h.at[b, pl.ds(base, _RPW)])
    pltpu.sync_copy(chamv.at[pl.ds(128, _RPW)], cham_h.at[b, pl.ds(base, _RPW)])


def _stats_body(val_ref, cham_ref, out_ref):
    v = val_ref[...]
    mean = jnp.mean(v, axis=1, keepdims=True)
    std = jnp.sqrt(jnp.sum((v - mean) ** 2, axis=1, keepdims=True)
                   / jnp.float32(_N - 1))
    thr = mean + _ALPHA * std
    w = (v > thr).astype(jnp.float32)
    knn = jnp.mean(jnp.mean(v * w, axis=1))
    l1 = jnp.mean(jnp.mean(cham_ref[...], axis=1))
    out_ref[...] = jnp.full((1, 128), l1 * _W1 + knn * _W2, jnp.float32)


@functools.partial(jax.jit, static_argnames=())
def kernel(adv_pc, ori_pc):
    ax = adv_pc[:, :, 0]
    ay = adv_pc[:, :, 1]
    az = adv_pc[:, :, 2]
    ox = ori_pc[:, :, 0]
    oy = ori_pc[:, :, 1]
    oz = ori_pc[:, :, 2]

    mesh = plsc.VectorSubcoreMesh(core_axis_name="c", subcore_axis_name="s")
    sc = functools.partial(
        pl.kernel,
        mesh=mesh,
        out_type=[
            jax.ShapeDtypeStruct((_B, _N), jnp.float32),
            jax.ShapeDtypeStruct((_B, _N), jnp.float32),
        ],
        scratch_types=[
            pltpu.VMEM((_N + 16,), jnp.float32),
            pltpu.VMEM((_N + 16,), jnp.float32),
            pltpu.VMEM((_N + 16,), jnp.float32),
            pltpu.VMEM((_N,), jnp.float32),
            pltpu.VMEM((_N,), jnp.float32),
            pltpu.VMEM((_N,), jnp.float32),
            pltpu.VMEM((_RPW + 128,), jnp.float32),
            pltpu.VMEM((_RPW + 128,), jnp.float32),
        ],
    )(_sc_body)
    value, cham = sc(ax, ay, az, ox, oy, oz)

    out = pl.pallas_call(
        _stats_body,
        out_shape=jax.ShapeDtypeStruct((1, 128), jnp.float32),
    )(value, cham)
    return out[0, 0]


# speedup vs baseline: 3.0573x; 3.0573x over previous
import functools

import jax
import jax.numpy as jnp
from jax.experimental import pallas as pl

_N = 1024
_NT = _N // 8
_KNN_K = 5
_ALPHA = 1.05
_W1 = 5.0
_W2 = 3.0
_BIG = 3.0e38
_DN = (((1,), (1,)), ((), ()))


def _body(m2_ref, apn_ref, aa_ref, aar_ref, m1_ref, oo_ref, out_ref):
    b = pl.program_id(0)
    m2 = m2_ref[0]
    apn = apn_ref[0]
    aa = aa_ref[0]
    aa_row = aar_ref[0]
    m1 = m1_ref[0]
    oo = oo_ref[0]

    inner2 = jax.lax.dot_general(m2, apn, _DN,
                                 preferred_element_type=jnp.float32)
    inner1 = jax.lax.dot_general(m1, apn, _DN,
                                 preferred_element_type=jnp.float32)

    cm = (aa_row + inner1[0:8, :]) + oo[0:8, :]
    R = [jnp.full((8, _N), _BIG, jnp.float32) for _ in range(6)]
    for k in range(_NT):
        x = (aa_row + inner2[k * 8:(k + 1) * 8, :]) + aa[k * 8:(k + 1) * 8, :]
        for j in range(5):
            mj = jnp.minimum(R[j], x)
            x = jnp.maximum(R[j], x)
            R[j] = mj
        R[5] = jnp.minimum(R[5], x)
        if k > 0:
            y = (aa_row + inner1[k * 8:(k + 1) * 8, :]) + oo[k * 8:(k + 1) * 8, :]
            cm = jnp.minimum(cm, y)

    l1 = jnp.mean(jnp.min(cm, axis=0))

    S = jnp.concatenate(R, axis=0)
    row = jax.lax.broadcasted_iota(jnp.int32, (48, _N), 0)
    acc = jnp.zeros((1, _N), jnp.float32)
    for j in range(_KNN_K + 1):
        m = jnp.min(S, axis=0, keepdims=True)
        if j > 0:
            acc = acc + m
        if j < _KNN_K:
            idx = jnp.min(jnp.where(S == m, row, 48), axis=0, keepdims=True)
            S = jnp.where(row == idx, _BIG, S)

    value = acc / jnp.float32(_KNN_K)
    mean = jnp.mean(value)
    std = jnp.sqrt(jnp.sum((value - mean) ** 2) / jnp.float32(_N - 1))
    thr = mean + _ALPHA * std
    w = (value > thr).astype(jnp.float32)
    knn = jnp.mean(value * w)

    part = (_W1 * l1 + _W2 * knn) * jnp.float32(1.0 / 8.0)

    @pl.when(b == 0)
    def _():
        out_ref[...] = jnp.zeros((1, 128), jnp.float32)

    out_ref[...] += jnp.full((1, 128), part, jnp.float32)


@functools.partial(jax.jit, static_argnames=())
def kernel(adv_pc, ori_pc):
    B = adv_pc.shape[0]
    aa = jnp.sum(adv_pc * adv_pc, axis=-1, keepdims=True)
    oo = jnp.sum(ori_pc * ori_pc, axis=-1, keepdims=True)
    zeros = jnp.zeros_like(adv_pc)
    z5 = jnp.concatenate([zeros, zeros[..., :2]], axis=-1)
    m2 = jnp.concatenate([-2.0 * adv_pc, z5], axis=-1)
    apn = jnp.concatenate([adv_pc, z5], axis=-1)
    m1 = jnp.concatenate([-2.0 * ori_pc, z5], axis=-1)
    aar = aa.reshape(B, 1, _N)

    out = pl.pallas_call(
        _body,
        grid=(B,),
        in_specs=[
            pl.BlockSpec((1, _N, 8), lambda b: (b, 0, 0)),
            pl.BlockSpec((1, _N, 8), lambda b: (b, 0, 0)),
            pl.BlockSpec((1, _N, 1), lambda b: (b, 0, 0)),
            pl.BlockSpec((1, 1, _N), lambda b: (b, 0, 0)),
            pl.BlockSpec((1, _N, 8), lambda b: (b, 0, 0)),
            pl.BlockSpec((1, _N, 1), lambda b: (b, 0, 0)),
        ],
        out_specs=pl.BlockSpec((1, 128), lambda b: (0, 0)),
        out_shape=jax.ShapeDtypeStruct((1, 128), jnp.float32),
    )(m2, apn, aa, aar, m1, oo)

    return out[0, 0]
